# baseline (speedup 1.0000x reference)
import functools

import jax
import jax.numpy as jnp
from jax import lax
from jax.experimental import pallas as pl
from jax.experimental.pallas import tpu as pltpu
from jax.experimental.pallas import tpu_sc as plsc

B = 16384
L = 20
N = B * L
D = 64
NUM_E = 1000000
INT_WEIGHT = 0.1
POP_WEIGHT = 0.1
DIS_PEN = 0.01

NC = 2
NS = 16
NW = NC * NS
NP = N // NW
C = 128
NCHUNK = NP // C

RS = (N // 128, 128)

_mesh = plsc.VectorSubcoreMesh(
    core_axis_name="c", subcore_axis_name="s", num_cores=NC, num_subcores=NS
)


def _iota16():
  return lax.iota(jnp.int32, 16)


_GDN = lax.GatherDimensionNumbers(
    offset_dims=(), collapsed_slice_dims=(0,), start_index_map=(0,))


def _lane_perm(x, perm):
  return lax.gather(x, perm[:, None], _GDN, slice_sizes=(1,),
                    mode=lax.GatherScatterMode.PROMISE_IN_BOUNDS)


def _hsum(x):
  iota = _iota16()
  for sh in (8, 4, 2, 1):
    x = x + _lane_perm(x, iota ^ sh)
  return x


def _dot4(a, b):
  return (a[0] * b[0] + a[1] * b[1]) + (a[2] * b[2] + a[3] * b[3])


def _sq4(a, b):
  d0 = a[0] - b[0]
  d1 = a[1] - b[1]
  d2 = a[2] - b[2]
  d3 = a[3] - b[3]
  return (d0 * d0 + d1 * d1) + (d2 * d2 + d3 * d3)


@functools.partial(
    pl.kernel,
    out_type=(
        jax.ShapeDtypeStruct((N,), jnp.float32),
        jax.ShapeDtypeStruct((N,), jnp.float32),
        jax.ShapeDtypeStruct((N,), jnp.float32),
        jax.ShapeDtypeStruct((N,), jnp.float32),
        jax.ShapeDtypeStruct((N,), jnp.float32),
        jax.ShapeDtypeStruct((N,), jnp.float32),
        jax.ShapeDtypeStruct((N,), jnp.float32),
        jax.ShapeDtypeStruct((NUM_E,), jnp.int32),
        jax.ShapeDtypeStruct((NUM_E,), jnp.int32),
    ),
    mesh=_mesh,
    compiler_params=pltpu.CompilerParams(needs_layout_passes=False,
                                         use_tc_tiling_on_sc=False),
    scratch_types=(
        [pltpu.VMEM((C,), jnp.int32)] * 6
        + [pltpu.VMEM((C, D), jnp.float32)] * 12
        + [pltpu.VMEM((C,), jnp.int32)] * 4
        + [pltpu.VMEM((C,), jnp.float32)] * 7
        + [pltpu.SemaphoreType.DMA] * 4
    ),
)
def _main_sc(uidx, pidx, nidx, t_uint, t_upop, t_iint, t_ipop,
             o_pint, o_nint, o_ppop, o_npop, o_d2p, o_d2n, o_d2u,
             o_tagi, o_tagu,
             v_ui0, v_pi0, v_ni0, v_ui1, v_pi1, v_ni1,
             ru0, rup0, rpi0, rpp0, rni0, rnp0,
             ru1, rup1, rpi1, rpp1, rni1, rnp1,
             v_pos0, v_posn0, v_pos1, v_posn1,
             s_pint, s_nint, s_ppop, s_npop, v_d2p, v_d2n, v_d2u,
             sem_g0, sem_g1, sem_s0, sem_s1):
  wid = lax.axis_index("s") * NC + lax.axis_index("c")
  iota = _iota16()
  lane15 = iota == 15

  idx_b = ((v_ui0, v_pi0, v_ni0), (v_ui1, v_pi1, v_ni1))
  rows_b = ((ru0, rup0, rpi0, rpp0, rni0, rnp0),
            (ru1, rup1, rpi1, rpp1, rni1, rnp1))
  pos_b = ((v_pos0, v_posn0), (v_pos1, v_posn1))
  sem_g = (sem_g0, sem_g1)
  sem_s = (sem_s0, sem_s1)

  def gather_descs(b):
    vu, vp, vn = idx_b[b]
    ru, rup, rpi, rpp, rni, rnp = rows_b[b]
    return ((t_uint.at[vu], ru), (t_upop.at[vu], rup),
            (t_iint.at[vp], rpi), (t_ipop.at[vp], rpp),
            (t_iint.at[vn], rni), (t_ipop.at[vn], rnp))

  def scatter_descs(b):
    vu, vp, vn = idx_b[b]
    vpos, vposn = pos_b[b]
    return ((vpos, o_tagi.at[vp]), (vposn, o_tagi.at[vn]),
            (vpos, o_tagu.at[vu]))

  def load_and_fire(b, base):
    vu, vp, vn = idx_b[b]
    pltpu.sync_copy(uidx.at[pl.ds(base, C)], vu)
    pltpu.sync_copy(pidx.at[pl.ds(base, C)], vp)
    pltpu.sync_copy(nidx.at[pl.ds(base, C)], vn)
    for src, dst in gather_descs(b):
      pltpu.async_copy(src, dst, sem_g[b])

  def wait_gathers(b):
    for src, dst in gather_descs(b):
      pltpu.make_async_copy(src, dst, sem_g[b]).wait()

  def fire_scatters(b):
    for src, dst in scatter_descs(b):
      pltpu.async_copy(src, dst, sem_s[b])

  def wait_scatters(b):
    for src, dst in scatter_descs(b):
      pltpu.make_async_copy(src, dst, sem_s[b]).wait()

  load_and_fire(0, wid * NP)

  def pair_body(j2, carry):
    for b in (0, 1):
      c = 2 * j2 + b
      base = wid * NP + c * C
      nb = 1 - b
      wait_gathers(b)

      @pl.when(c >= 1)
      def _():
        wait_scatters(nb)

      @pl.when(c + 1 < NCHUNK)
      def _():
        load_and_fire(nb, base + C)

      vpos, vposn = pos_b[b]

      def pos_body(k, cc):
        v = iota + jnp.full((16,), base + k * 16, jnp.int32)
        vpos[pl.ds(k * 16, 16)] = v
        vposn[pl.ds(k * 16, 16)] = v + N
        return cc
      lax.fori_loop(0, C // 16, pos_body, 0)

      ru, rup, rpi, rpp, rni, rnp = rows_b[b]

      def row_body(r, cc):
        ui = [ru[r, pl.ds(k * 16, 16)] for k in range(4)]
        up = [rup[r, pl.ds(k * 16, 16)] for k in range(4)]
        pi_ = [rpi[r, pl.ds(k * 16, 16)] for k in range(4)]
        pp = [rpp[r, pl.ds(k * 16, 16)] for k in range(4)]
        ni_ = [rni[r, pl.ds(k * 16, 16)] for k in range(4)]
        np_ = [rnp[r, pl.ds(k * 16, 16)] for k in range(4)]
        ridx = jnp.full((16,), r, jnp.int32)

        def red_store(vec, ref):
          plsc.store_scatter(ref, [ridx], _hsum(vec), mask=lane15)

        red_store(_dot4(ui, pi_), s_pint)
        red_store(_dot4(ui, ni_), s_nint)
        red_store(_dot4(up, pp), s_ppop)
        red_store(_dot4(up, np_), s_npop)
        red_store(_sq4(pi_, pp), v_d2p)
        red_store(_sq4(ni_, np_), v_d2n)
        red_store(_sq4(ui, up), v_d2u)
        return cc
      lax.fori_loop(0, C, row_body, 0)

      pltpu.sync_copy(s_pint, o_pint.at[pl.ds(base, C)])
      pltpu.sync_copy(s_nint, o_nint.at[pl.ds(base, C)])
      pltpu.sync_copy(s_ppop, o_ppop.at[pl.ds(base, C)])
      pltpu.sync_copy(s_npop, o_npop.at[pl.ds(base, C)])
      pltpu.sync_copy(v_d2p, o_d2p.at[pl.ds(base, C)])
      pltpu.sync_copy(v_d2n, o_d2n.at[pl.ds(base, C)])
      pltpu.sync_copy(v_d2u, o_d2u.at[pl.ds(base, C)])

      fire_scatters(b)
    return carry

  lax.fori_loop(0, NCHUNK // 2, pair_body, 0)
  wait_scatters(1)


@functools.partial(
    pl.kernel,
    out_type=jax.ShapeDtypeStruct((NW * 8,), jnp.float32),
    mesh=_mesh,
    compiler_params=pltpu.CompilerParams(needs_layout_passes=False,
                                         use_tc_tiling_on_sc=False),
    scratch_types=(
        [pltpu.VMEM((C,), jnp.int32)] * 12
        + [pltpu.VMEM((C,), jnp.float32)] * 6
        + [pltpu.VMEM((16,), jnp.float32)] * 5
        + [pltpu.SemaphoreType.DMA] * 2
    ),
)
def _reduce_sc(uidx, pidx, nidx, d2p, d2n, d2u, tagi, tagu,
               o_part,
               v_ui0, v_pi0, v_ni0, v_tu0, v_tp0, v_tn0,
               v_ui1, v_pi1, v_ni1, v_tu1, v_tp1, v_tn1,
               v_d2p0, v_d2n0, v_d2u0, v_d2p1, v_d2n1, v_d2u1,
               a_isum, a_icnt, a_usum, a_ucnt, v_out, sem0, sem1):
  wid = lax.axis_index("s") * NC + lax.axis_index("c")
  iota = _iota16()
  lane15 = iota == 15
  zeros = jnp.zeros((16,), jnp.float32)
  ones = jnp.ones((16,), jnp.float32)
  a_isum[...] = zeros
  a_icnt[...] = zeros
  a_usum[...] = zeros
  a_ucnt[...] = zeros

  idx_b = ((v_ui0, v_pi0, v_ni0), (v_ui1, v_pi1, v_ni1))
  tag_b = ((v_tu0, v_tp0, v_tn0), (v_tu1, v_tp1, v_tn1))
  d2_b = ((v_d2p0, v_d2n0, v_d2u0), (v_d2p1, v_d2n1, v_d2u1))
  sems = (sem0, sem1)

  def tag_descs(b):
    vu, vp, vn = idx_b[b]
    tu, tp, tn = tag_b[b]
    return ((tagu.at[vu], tu), (tagi.at[vp], tp), (tagi.at[vn], tn))

  def load_and_fire(b, base):
    vu, vp, vn = idx_b[b]
    vdp, vdn, vdu = d2_b[b]
    pltpu.sync_copy(uidx.at[pl.ds(base, C)], vu)
    pltpu.sync_copy(pidx.at[pl.ds(base, C)], vp)
    pltpu.sync_copy(nidx.at[pl.ds(base, C)], vn)
    pltpu.sync_copy(d2p.at[pl.ds(base, C)], vdp)
    pltpu.sync_copy(d2n.at[pl.ds(base, C)], vdn)
    pltpu.sync_copy(d2u.at[pl.ds(base, C)], vdu)
    for src, dst in tag_descs(b):
      pltpu.async_copy(src, dst, sems[b])

  def wait_tags(b):
    for src, dst in tag_descs(b):
      pltpu.make_async_copy(src, dst, sems[b]).wait()

  load_and_fire(0, wid * NP)

  def pair_body(j2, carry):
    for b in (0, 1):
      c = 2 * j2 + b
      base = wid * NP + c * C
      wait_tags(b)

      @pl.when(c + 1 < NCHUNK)
      def _():
        load_and_fire(1 - b, base + C)

      v_tu, v_tp, v_tn = tag_b[b]
      v_d2p, v_d2n, v_d2u = d2_b[b]

      def grp_body(k, cc):
        pos = iota + jnp.full((16,), base + k * 16, jnp.int32)
        sl = pl.ds(k * 16, 16)
        m_p = v_tp[sl] == pos
        m_n = v_tn[sl] == (pos + N)
        m_u = v_tu[sl] == pos
        a_isum[...] = a_isum[...] + (jnp.where(m_p, v_d2p[sl], zeros)
                                     + jnp.where(m_n, v_d2n[sl], zeros))
        a_icnt[...] = a_icnt[...] + (jnp.where(m_p, ones, zeros)
                                     + jnp.where(m_n, ones, zeros))
        a_usum[...] = a_usum[...] + jnp.where(m_u, v_d2u[sl], zeros)
        a_ucnt[...] = a_ucnt[...] + jnp.where(m_u, ones, zeros)
        return cc
      lax.fori_loop(0, C // 16, grp_body, 0)
    return carry

  lax.fori_loop(0, NCHUNK // 2, pair_body, 0)

  v_out[...] = jnp.zeros((16,), jnp.float32)
  for slot, ref in enumerate((a_isum, a_icnt, a_usum, a_ucnt)):
    plsc.store_scatter(v_out, [jnp.full((16,), slot, jnp.int32)],
                       _hsum(ref[...]), mask=lane15)
  pltpu.sync_copy(v_out.at[pl.ds(0, 8)], o_part.at[pl.ds(wid * 8, 8)])


def _tc_loss_body(pint_ref, ppop_ref, nint_ref, npop_ref, maskf_ref,
                  part_ref, out_ref):
  pi = pint_ref[...]
  pp = ppop_ref[...]
  ni = nint_ref[...]
  np_ = npop_ref[...]
  mf = maskf_ref[...]
  nmf = 1.0 - mf

  def logsig(x):
    return jnp.log(jax.nn.sigmoid(x))

  loss_int = -jnp.mean(mf * logsig(pi - ni))
  loss_pop = (-jnp.mean(mf * logsig(np_ - pp))
              - jnp.mean(nmf * logsig(pp - np_)))
  loss_total = -jnp.mean(logsig((pi + pp) - (ni + np_)))

  ss = jnp.sum(part_ref[...], axis=0, keepdims=True)
  item_mse = ss[0, 0] / (ss[0, 1] * D)
  user_mse = ss[0, 2] / (ss[0, 3] * D)

  loss = (INT_WEIGHT * loss_int + POP_WEIGHT * loss_pop + loss_total
          - DIS_PEN * (item_mse + user_mse))
  out_ref[...] = jnp.reshape(loss, (1, 1))


def kernel(user, item_p, item_n, mask, users_int, users_pop, items_int,
           items_pop):
  uf = user.reshape(-1).astype(jnp.int32)
  pf = item_p.reshape(-1).astype(jnp.int32)
  nf = item_n.reshape(-1).astype(jnp.int32)

  (p_int, n_int, p_pop, n_pop, d2p, d2n, d2u, tagi, tagu) = _main_sc(
      uf, pf, nf, users_int, users_pop, items_int, items_pop)

  partials = _reduce_sc(uf, pf, nf, d2p, d2n, d2u, tagi, tagu)

  maskf = mask.reshape(RS).astype(jnp.float32)
  lossm = pl.pallas_call(
      _tc_loss_body,
      out_shape=jax.ShapeDtypeStruct((1, 1), jnp.float32),
  )(p_int.reshape(RS), p_pop.reshape(RS), n_int.reshape(RS),
    n_pop.reshape(RS), maskf, partials.reshape(NW, 8))
  loss = lossm[0, 0]

  return (loss, p_int.reshape(B, L), p_pop.reshape(B, L),
          n_int.reshape(B, L), n_pop.reshape(B, L))


# speedup vs baseline: 1.2014x; 1.2014x over previous
import functools

import jax
import jax.numpy as jnp
from jax import lax
from jax.experimental import pallas as pl
from jax.experimental.pallas import tpu as pltpu
from jax.experimental.pallas import tpu_sc as plsc

B = 16384
L = 20
N = B * L
D = 64
NUM_E = 1000000
INT_WEIGHT = 0.1
POP_WEIGHT = 0.1
DIS_PEN = 0.01

NC = 2
NS = 16
NW = NC * NS
NP = N // NW
C = 128
NCHUNK = NP // C

RS = (N // 128, 128)

_mesh = plsc.VectorSubcoreMesh(
    core_axis_name="c", subcore_axis_name="s", num_cores=NC, num_subcores=NS
)


def _iota16():
  return lax.iota(jnp.int32, 16)


_GDN = lax.GatherDimensionNumbers(
    offset_dims=(), collapsed_slice_dims=(0,), start_index_map=(0,))


def _lane_perm(x, perm):
  return lax.gather(x, perm[:, None], _GDN, slice_sizes=(1,),
                    mode=lax.GatherScatterMode.PROMISE_IN_BOUNDS)


def _hsum(x):
  iota = _iota16()
  for sh in (8, 4, 2, 1):
    x = x + _lane_perm(x, iota ^ sh)
  return x


def _dot4(a, b):
  acc = a[0] * b[0]
  acc = acc + a[1] * b[1]
  acc = acc + a[2] * b[2]
  acc = acc + a[3] * b[3]
  return acc


def _sq4(a, b):
  d0 = a[0] - b[0]
  acc = d0 * d0
  d1 = a[1] - b[1]
  acc = acc + d1 * d1
  d2 = a[2] - b[2]
  acc = acc + d2 * d2
  d3 = a[3] - b[3]
  acc = acc + d3 * d3
  return acc


@functools.partial(
    pl.kernel,
    out_type=(
        jax.ShapeDtypeStruct((N,), jnp.float32),
        jax.ShapeDtypeStruct((N,), jnp.float32),
        jax.ShapeDtypeStruct((N,), jnp.float32),
        jax.ShapeDtypeStruct((N,), jnp.float32),
        jax.ShapeDtypeStruct((N,), jnp.float32),
        jax.ShapeDtypeStruct((N,), jnp.float32),
        jax.ShapeDtypeStruct((N,), jnp.float32),
        jax.ShapeDtypeStruct((NUM_E,), jnp.int32),
        jax.ShapeDtypeStruct((NUM_E,), jnp.int32),
    ),
    mesh=_mesh,
    compiler_params=pltpu.CompilerParams(needs_layout_passes=False,
                                         use_tc_tiling_on_sc=False),
    scratch_types=(
        [pltpu.VMEM((C,), jnp.int32)] * 6
        + [pltpu.VMEM((C, D), jnp.float32)] * 12
        + [pltpu.VMEM((C,), jnp.int32)] * 4
        + [pltpu.VMEM((C,), jnp.float32)] * 7
        + [pltpu.VMEM((256,), jnp.float32)] * 7
        + [pltpu.SemaphoreType.DMA] * 4
    ),
)
def _main_sc(uidx, pidx, nidx, t_uint, t_upop, t_iint, t_ipop,
             o_pint, o_nint, o_ppop, o_npop, o_d2p, o_d2n, o_d2u,
             o_tagi, o_tagu,
             v_ui0, v_pi0, v_ni0, v_ui1, v_pi1, v_ni1,
             ru0, rup0, rpi0, rpp0, rni0, rnp0,
             ru1, rup1, rpi1, rpp1, rni1, rnp1,
             v_pos0, v_posn0, v_pos1, v_posn1,
             s_pint, s_nint, s_ppop, s_npop, v_d2p, v_d2n, v_d2u,
             m0, m1, m2, m3, m4, m5, m6,
             sem_g0, sem_g1, sem_s0, sem_s1):
  wid = lax.axis_index("s") * NC + lax.axis_index("c")
  iota = _iota16()
  mats = (m0, m1, m2, m3, m4, m5, m6)
  stages = (s_pint, s_nint, s_ppop, s_npop, v_d2p, v_d2n, v_d2u)
  colidx = [iota * 16 + c for c in range(16)]

  idx_b = ((v_ui0, v_pi0, v_ni0), (v_ui1, v_pi1, v_ni1))
  rows_b = ((ru0, rup0, rpi0, rpp0, rni0, rnp0),
            (ru1, rup1, rpi1, rpp1, rni1, rnp1))
  pos_b = ((v_pos0, v_posn0), (v_pos1, v_posn1))
  sem_g = (sem_g0, sem_g1)
  sem_s = (sem_s0, sem_s1)

  def gather_descs(b):
    vu, vp, vn = idx_b[b]
    ru, rup, rpi, rpp, rni, rnp = rows_b[b]
    return ((t_uint.at[vu], ru), (t_upop.at[vu], rup),
            (t_iint.at[vp], rpi), (t_ipop.at[vp], rpp),
            (t_iint.at[vn], rni), (t_ipop.at[vn], rnp))

  def scatter_descs(b):
    vu, vp, vn = idx_b[b]
    vpos, vposn = pos_b[b]
    return ((vpos, o_tagi.at[vp]), (vposn, o_tagi.at[vn]),
            (vpos, o_tagu.at[vu]))

  def load_and_fire(b, base):
    vu, vp, vn = idx_b[b]
    pltpu.sync_copy(uidx.at[pl.ds(base, C)], vu)
    pltpu.sync_copy(pidx.at[pl.ds(base, C)], vp)
    pltpu.sync_copy(nidx.at[pl.ds(base, C)], vn)
    for src, dst in gather_descs(b):
      pltpu.async_copy(src, dst, sem_g[b])

  def wait_gathers(b):
    for src, dst in gather_descs(b):
      pltpu.make_async_copy(src, dst, sem_g[b]).wait()

  def fire_scatters(b):
    for src, dst in scatter_descs(b):
      pltpu.async_copy(src, dst, sem_s[b])

  def wait_scatters(b):
    for src, dst in scatter_descs(b):
      pltpu.make_async_copy(src, dst, sem_s[b]).wait()

  load_and_fire(0, wid * NP)

  def pair_body(j2, carry):
    for b in (0, 1):
      c = 2 * j2 + b
      base = wid * NP + c * C
      nb = 1 - b
      wait_gathers(b)

      @pl.when(c >= 1)
      def _():
        wait_scatters(nb)

      @pl.when(c + 1 < NCHUNK)
      def _():
        load_and_fire(nb, base + C)

      vpos, vposn = pos_b[b]

      def pos_body(k, cc):
        v = iota + jnp.full((16,), base + k * 16, jnp.int32)
        vpos[pl.ds(k * 16, 16)] = v
        vposn[pl.ds(k * 16, 16)] = v + N
        return cc
      lax.fori_loop(0, C // 16, pos_body, 0)

      ru, rup, rpi, rpp, rni, rnp = rows_b[b]

      def grp_body(g, cc):
        for r in range(16):
          row = g * 16 + r
          ui = [ru[row, pl.ds(k * 16, 16)] for k in range(4)]
          up = [rup[row, pl.ds(k * 16, 16)] for k in range(4)]
          pi_ = [rpi[row, pl.ds(k * 16, 16)] for k in range(4)]
          pp = [rpp[row, pl.ds(k * 16, 16)] for k in range(4)]
          ni_ = [rni[row, pl.ds(k * 16, 16)] for k in range(4)]
          np_ = [rnp[row, pl.ds(k * 16, 16)] for k in range(4)]
          sl = pl.ds(16 * r, 16)
          m0[sl] = _dot4(ui, pi_)
          m1[sl] = _dot4(ui, ni_)
          m2[sl] = _dot4(up, pp)
          m3[sl] = _dot4(up, np_)
          m4[sl] = _sq4(pi_, pp)
          m5[sl] = _sq4(ni_, np_)
          m6[sl] = _sq4(ui, up)
        for q in range(7):
          acc = plsc.load_gather(mats[q], [colidx[0]])
          for c in range(1, 16):
            acc = acc + plsc.load_gather(mats[q], [colidx[c]])
          stages[q][pl.ds(g * 16, 16)] = acc
        return cc
      lax.fori_loop(0, C // 16, grp_body, 0)

      pltpu.sync_copy(s_pint, o_pint.at[pl.ds(base, C)])
      pltpu.sync_copy(s_nint, o_nint.at[pl.ds(base, C)])
      pltpu.sync_copy(s_ppop, o_ppop.at[pl.ds(base, C)])
      pltpu.sync_copy(s_npop, o_npop.at[pl.ds(base, C)])
      pltpu.sync_copy(v_d2p, o_d2p.at[pl.ds(base, C)])
      pltpu.sync_copy(v_d2n, o_d2n.at[pl.ds(base, C)])
      pltpu.sync_copy(v_d2u, o_d2u.at[pl.ds(base, C)])

      fire_scatters(b)
    return carry

  lax.fori_loop(0, NCHUNK // 2, pair_body, 0)
  wait_scatters(1)


@functools.partial(
    pl.kernel,
    out_type=jax.ShapeDtypeStruct((NW * 8,), jnp.float32),
    mesh=_mesh,
    compiler_params=pltpu.CompilerParams(needs_layout_passes=False,
                                         use_tc_tiling_on_sc=False),
    scratch_types=(
        [pltpu.VMEM((C,), jnp.int32)] * 6
        + [pltpu.VMEM((C,), jnp.float32)] * 3
        + [pltpu.VMEM((16,), jnp.float32)] * 5
        + [pltpu.SemaphoreType.DMA]
    ),
)
def _reduce_sc(uidx, pidx, nidx, d2p, d2n, d2u, tagi, tagu,
               o_part,
               v_ui, v_pi, v_ni, v_tu, v_tp, v_tn, v_d2p, v_d2n, v_d2u,
               a_isum, a_icnt, a_usum, a_ucnt, v_out, sem):
  wid = lax.axis_index("s") * NC + lax.axis_index("c")
  iota = _iota16()
  lane15 = iota == 15
  zeros = jnp.zeros((16,), jnp.float32)
  ones = jnp.ones((16,), jnp.float32)
  a_isum[...] = zeros
  a_icnt[...] = zeros
  a_usum[...] = zeros
  a_ucnt[...] = zeros

  def chunk_body(j, carry):
    base = wid * NP + j * C
    pltpu.sync_copy(uidx.at[pl.ds(base, C)], v_ui)
    pltpu.sync_copy(pidx.at[pl.ds(base, C)], v_pi)
    pltpu.sync_copy(nidx.at[pl.ds(base, C)], v_ni)
    pltpu.sync_copy(d2p.at[pl.ds(base, C)], v_d2p)
    pltpu.sync_copy(d2n.at[pl.ds(base, C)], v_d2n)
    pltpu.sync_copy(d2u.at[pl.ds(base, C)], v_d2u)
    c1 = pltpu.async_copy(tagu.at[v_ui], v_tu, sem)
    c2 = pltpu.async_copy(tagi.at[v_pi], v_tp, sem)
    c3 = pltpu.async_copy(tagi.at[v_ni], v_tn, sem)
    c1.wait()
    c2.wait()
    c3.wait()

    def grp_body(k, cc):
      pos = iota + jnp.full((16,), base + k * 16, jnp.int32)
      sl = pl.ds(k * 16, 16)
      m_p = v_tp[sl] == pos
      m_n = v_tn[sl] == (pos + N)
      m_u = v_tu[sl] == pos
      a_isum[...] = a_isum[...] + (jnp.where(m_p, v_d2p[sl], zeros)
                                   + jnp.where(m_n, v_d2n[sl], zeros))
      a_icnt[...] = a_icnt[...] + (jnp.where(m_p, ones, zeros)
                                   + jnp.where(m_n, ones, zeros))
      a_usum[...] = a_usum[...] + jnp.where(m_u, v_d2u[sl], zeros)
      a_ucnt[...] = a_ucnt[...] + jnp.where(m_u, ones, zeros)
      return cc
    lax.fori_loop(0, C // 16, grp_body, 0)
    return carry

  lax.fori_loop(0, NCHUNK, chunk_body, 0)

  v_out[...] = jnp.zeros((16,), jnp.float32)
  for slot, ref in enumerate((a_isum, a_icnt, a_usum, a_ucnt)):
    plsc.store_scatter(v_out, [jnp.full((16,), slot, jnp.int32)],
                       _hsum(ref[...]), mask=lane15)
  pltpu.sync_copy(v_out.at[pl.ds(0, 8)], o_part.at[pl.ds(wid * 8, 8)])


def _tc_loss_body(pint_ref, ppop_ref, nint_ref, npop_ref, maskf_ref,
                  part_ref, out_ref):
  pi = pint_ref[...]
  pp = ppop_ref[...]
  ni = nint_ref[...]
  np_ = npop_ref[...]
  mf = maskf_ref[...]
  nmf = 1.0 - mf

  def logsig(x):
    return jnp.log(jax.nn.sigmoid(x))

  loss_int = -jnp.mean(mf * logsig(pi - ni))
  loss_pop = (-jnp.mean(mf * logsig(np_ - pp))
              - jnp.mean(nmf * logsig(pp - np_)))
  loss_total = -jnp.mean(logsig((pi + pp) - (ni + np_)))

  ss = jnp.sum(part_ref[...], axis=0, keepdims=True)
  item_mse = ss[0, 0] / (ss[0, 1] * D)
  user_mse = ss[0, 2] / (ss[0, 3] * D)

  loss = (INT_WEIGHT * loss_int + POP_WEIGHT * loss_pop + loss_total
          - DIS_PEN * (item_mse + user_mse))
  out_ref[...] = jnp.reshape(loss, (1, 1))


def kernel(user, item_p, item_n, mask, users_int, users_pop, items_int,
           items_pop):
  uf = user.reshape(-1).astype(jnp.int32)
  pf = item_p.reshape(-1).astype(jnp.int32)
  nf = item_n.reshape(-1).astype(jnp.int32)

  (p_int, n_int, p_pop, n_pop, d2p, d2n, d2u, tagi, tagu) = _main_sc(
      uf, pf, nf, users_int, users_pop, items_int, items_pop)

  partials = _reduce_sc(uf, pf, nf, d2p, d2n, d2u, tagi, tagu)

  maskf = mask.reshape(RS).astype(jnp.float32)
  lossm = pl.pallas_call(
      _tc_loss_body,
      out_shape=jax.ShapeDtypeStruct((1, 1), jnp.float32),
  )(p_int.reshape(RS), p_pop.reshape(RS), n_int.reshape(RS),
    n_pop.reshape(RS), maskf, partials.reshape(NW, 8))
  loss = lossm[0, 0]

  return (loss, p_int.reshape(B, L), p_pop.reshape(B, L),
          n_int.reshape(B, L), n_pop.reshape(B, L))


# speedup vs baseline: 1.2715x; 1.0584x over previous
import functools

import jax
import jax.numpy as jnp
from jax import lax
from jax.experimental import pallas as pl
from jax.experimental.pallas import tpu as pltpu
from jax.experimental.pallas import tpu_sc as plsc

B = 16384
L = 20
N = B * L
D = 64
NUM_E = 1000000
INT_WEIGHT = 0.1
POP_WEIGHT = 0.1
DIS_PEN = 0.01

NC = 2
NS = 16
NW = NC * NS
NP = N // NW
C = 128
NCHUNK = NP // C
CR = 512

RS = (N // 128, 128)

_mesh = plsc.VectorSubcoreMesh(
    core_axis_name="c", subcore_axis_name="s", num_cores=NC, num_subcores=NS
)


def _iota16():
  return lax.iota(jnp.int32, 16)


_GDN = lax.GatherDimensionNumbers(
    offset_dims=(), collapsed_slice_dims=(0,), start_index_map=(0,))


def _lane_perm(x, perm):
  return lax.gather(x, perm[:, None], _GDN, slice_sizes=(1,),
                    mode=lax.GatherScatterMode.PROMISE_IN_BOUNDS)


def _hsum(x):
  iota = _iota16()
  for sh in (8, 4, 2, 1):
    x = x + _lane_perm(x, iota ^ sh)
  return x


def _dot4(a, b):
  acc = a[0] * b[0]
  acc = acc + a[1] * b[1]
  acc = acc + a[2] * b[2]
  acc = acc + a[3] * b[3]
  return acc


def _sq4(a, b):
  d0 = a[0] - b[0]
  acc = d0 * d0
  d1 = a[1] - b[1]
  acc = acc + d1 * d1
  d2 = a[2] - b[2]
  acc = acc + d2 * d2
  d3 = a[3] - b[3]
  acc = acc + d3 * d3
  return acc


@functools.partial(
    pl.kernel,
    out_type=(
        jax.ShapeDtypeStruct((N,), jnp.float32),
        jax.ShapeDtypeStruct((N,), jnp.float32),
        jax.ShapeDtypeStruct((N,), jnp.float32),
        jax.ShapeDtypeStruct((N,), jnp.float32),
        jax.ShapeDtypeStruct((N,), jnp.float32),
        jax.ShapeDtypeStruct((N,), jnp.float32),
        jax.ShapeDtypeStruct((N,), jnp.float32),
        jax.ShapeDtypeStruct((NUM_E,), jnp.int32),
        jax.ShapeDtypeStruct((NUM_E,), jnp.int32),
    ),
    mesh=_mesh,
    compiler_params=pltpu.CompilerParams(needs_layout_passes=False,
                                         use_tc_tiling_on_sc=False),
    scratch_types=(
        [pltpu.VMEM((C,), jnp.int32)] * 6
        + [pltpu.VMEM((C, D), jnp.float32)] * 12
        + [pltpu.VMEM((C,), jnp.int32)] * 4
        + [pltpu.VMEM((C,), jnp.float32)] * 7
        + [pltpu.VMEM((256,), jnp.float32)] * 7
        + [pltpu.SemaphoreType.DMA] * 4
    ),
)
def _main_sc(uidx, pidx, nidx, t_uint, t_upop, t_iint, t_ipop,
             o_pint, o_nint, o_ppop, o_npop, o_d2p, o_d2n, o_d2u,
             o_tagi, o_tagu,
             v_ui0, v_pi0, v_ni0, v_ui1, v_pi1, v_ni1,
             ru0, rup0, rpi0, rpp0, rni0, rnp0,
             ru1, rup1, rpi1, rpp1, rni1, rnp1,
             v_pos0, v_posn0, v_pos1, v_posn1,
             s_pint, s_nint, s_ppop, s_npop, v_d2p, v_d2n, v_d2u,
             m0, m1, m2, m3, m4, m5, m6,
             sem_g0, sem_g1, sem_s0, sem_s1):
  wid = lax.axis_index("s") * NC + lax.axis_index("c")
  iota = _iota16()
  mats = (m0, m1, m2, m3, m4, m5, m6)
  stages = (s_pint, s_nint, s_ppop, s_npop, v_d2p, v_d2n, v_d2u)
  colidx = [iota * 16 + c for c in range(16)]

  idx_b = ((v_ui0, v_pi0, v_ni0), (v_ui1, v_pi1, v_ni1))
  rows_b = ((ru0, rup0, rpi0, rpp0, rni0, rnp0),
            (ru1, rup1, rpi1, rpp1, rni1, rnp1))
  pos_b = ((v_pos0, v_posn0), (v_pos1, v_posn1))
  sem_g = (sem_g0, sem_g1)
  sem_s = (sem_s0, sem_s1)

  def gather_descs(b):
    vu, vp, vn = idx_b[b]
    ru, rup, rpi, rpp, rni, rnp = rows_b[b]
    return ((t_uint.at[vu], ru), (t_upop.at[vu], rup),
            (t_iint.at[vp], rpi), (t_ipop.at[vp], rpp),
            (t_iint.at[vn], rni), (t_ipop.at[vn], rnp))

  def scatter_descs(b):
    vu, vp, vn = idx_b[b]
    vpos, vposn = pos_b[b]
    return ((vpos, o_tagi.at[vp]), (vposn, o_tagi.at[vn]),
            (vpos, o_tagu.at[vu]))

  def load_and_fire(b, base):
    vu, vp, vn = idx_b[b]
    pltpu.sync_copy(uidx.at[pl.ds(base, C)], vu)
    pltpu.sync_copy(pidx.at[pl.ds(base, C)], vp)
    pltpu.sync_copy(nidx.at[pl.ds(base, C)], vn)
    for src, dst in gather_descs(b):
      pltpu.async_copy(src, dst, sem_g[b])

  def wait_gathers(b):
    for src, dst in gather_descs(b):
      pltpu.make_async_copy(src, dst, sem_g[b]).wait()

  def fire_scatters(b):
    for src, dst in scatter_descs(b):
      pltpu.async_copy(src, dst, sem_s[b])

  def wait_scatters(b):
    for src, dst in scatter_descs(b):
      pltpu.make_async_copy(src, dst, sem_s[b]).wait()

  load_and_fire(0, wid * NP)

  def pair_body(j2, carry):
    for b in (0, 1):
      c = 2 * j2 + b
      base = wid * NP + c * C
      nb = 1 - b
      wait_gathers(b)

      @pl.when(c >= 1)
      def _():
        wait_scatters(nb)

      @pl.when(c + 1 < NCHUNK)
      def _():
        load_and_fire(nb, base + C)

      vpos, vposn = pos_b[b]

      def pos_body(k, cc):
        v = iota + jnp.full((16,), base + k * 16, jnp.int32)
        vpos[pl.ds(k * 16, 16)] = v
        vposn[pl.ds(k * 16, 16)] = v + N
        return cc
      lax.fori_loop(0, C // 16, pos_body, 0)

      ru, rup, rpi, rpp, rni, rnp = rows_b[b]

      def grp_body(g, cc):
        for r in range(16):
          row = g * 16 + r
          ui = [ru[row, pl.ds(k * 16, 16)] for k in range(4)]
          up = [rup[row, pl.ds(k * 16, 16)] for k in range(4)]
          pi_ = [rpi[row, pl.ds(k * 16, 16)] for k in range(4)]
          pp = [rpp[row, pl.ds(k * 16, 16)] for k in range(4)]
          ni_ = [rni[row, pl.ds(k * 16, 16)] for k in range(4)]
          np_ = [rnp[row, pl.ds(k * 16, 16)] for k in range(4)]
          sl = pl.ds(16 * r, 16)
          m0[sl] = _dot4(ui, pi_)
          m1[sl] = _dot4(ui, ni_)
          m2[sl] = _dot4(up, pp)
          m3[sl] = _dot4(up, np_)
          m4[sl] = _sq4(pi_, pp)
          m5[sl] = _sq4(ni_, np_)
          m6[sl] = _sq4(ui, up)
        for q in range(7):
          acc = plsc.load_gather(mats[q], [colidx[0]])
          for c in range(1, 16):
            acc = acc + plsc.load_gather(mats[q], [colidx[c]])
          stages[q][pl.ds(g * 16, 16)] = acc
        return cc
      lax.fori_loop(0, C // 16, grp_body, 0)

      pltpu.sync_copy(s_pint, o_pint.at[pl.ds(base, C)])
      pltpu.sync_copy(s_nint, o_nint.at[pl.ds(base, C)])
      pltpu.sync_copy(s_ppop, o_ppop.at[pl.ds(base, C)])
      pltpu.sync_copy(s_npop, o_npop.at[pl.ds(base, C)])
      pltpu.sync_copy(v_d2p, o_d2p.at[pl.ds(base, C)])
      pltpu.sync_copy(v_d2n, o_d2n.at[pl.ds(base, C)])
      pltpu.sync_copy(v_d2u, o_d2u.at[pl.ds(base, C)])

      fire_scatters(b)
    return carry

  lax.fori_loop(0, NCHUNK // 2, pair_body, 0)
  wait_scatters(1)


@functools.partial(
    pl.kernel,
    out_type=jax.ShapeDtypeStruct((NW * 8,), jnp.float32),
    mesh=_mesh,
    compiler_params=pltpu.CompilerParams(needs_layout_passes=False,
                                         use_tc_tiling_on_sc=False),
    scratch_types=(
        [pltpu.VMEM((CR,), jnp.int32)] * 6
        + [pltpu.VMEM((CR,), jnp.float32)] * 3
        + [pltpu.VMEM((16,), jnp.float32)] * 5
        + [pltpu.SemaphoreType.DMA]
    ),
)
def _reduce_sc(uidx, pidx, nidx, d2p, d2n, d2u, tagi, tagu,
               o_part,
               v_ui, v_pi, v_ni, v_tu, v_tp, v_tn, v_d2p, v_d2n, v_d2u,
               a_isum, a_icnt, a_usum, a_ucnt, v_out, sem):
  wid = lax.axis_index("s") * NC + lax.axis_index("c")
  iota = _iota16()
  lane15 = iota == 15
  zeros = jnp.zeros((16,), jnp.float32)
  ones = jnp.ones((16,), jnp.float32)
  a_isum[...] = zeros
  a_icnt[...] = zeros
  a_usum[...] = zeros
  a_ucnt[...] = zeros

  def chunk_body(j, carry):
    base = wid * NP + j * CR
    pltpu.sync_copy(uidx.at[pl.ds(base, CR)], v_ui)
    pltpu.sync_copy(pidx.at[pl.ds(base, CR)], v_pi)
    pltpu.sync_copy(nidx.at[pl.ds(base, CR)], v_ni)
    pltpu.sync_copy(d2p.at[pl.ds(base, CR)], v_d2p)
    pltpu.sync_copy(d2n.at[pl.ds(base, CR)], v_d2n)
    pltpu.sync_copy(d2u.at[pl.ds(base, CR)], v_d2u)
    cps = []
    for r in range(CR // 128):
      sl128 = pl.ds(128 * r, 128)
      cps.append(pltpu.async_copy(tagu.at[v_ui.at[sl128]], v_tu.at[sl128],
                                  sem))
      cps.append(pltpu.async_copy(tagi.at[v_pi.at[sl128]], v_tp.at[sl128],
                                  sem))
      cps.append(pltpu.async_copy(tagi.at[v_ni.at[sl128]], v_tn.at[sl128],
                                  sem))
    for cp in cps:
      cp.wait()

    def grp_body(k, cc):
      pos = iota + jnp.full((16,), base + k * 16, jnp.int32)
      sl = pl.ds(k * 16, 16)
      m_p = v_tp[sl] == pos
      m_n = v_tn[sl] == (pos + N)
      m_u = v_tu[sl] == pos
      a_isum[...] = a_isum[...] + (jnp.where(m_p, v_d2p[sl], zeros)
                                   + jnp.where(m_n, v_d2n[sl], zeros))
      a_icnt[...] = a_icnt[...] + (jnp.where(m_p, ones, zeros)
                                   + jnp.where(m_n, ones, zeros))
      a_usum[...] = a_usum[...] + jnp.where(m_u, v_d2u[sl], zeros)
      a_ucnt[...] = a_ucnt[...] + jnp.where(m_u, ones, zeros)
      return cc
    lax.fori_loop(0, CR // 16, grp_body, 0)
    return carry

  lax.fori_loop(0, NP // CR, chunk_body, 0)

  v_out[...] = jnp.zeros((16,), jnp.float32)
  for slot, ref in enumerate((a_isum, a_icnt, a_usum, a_ucnt)):
    plsc.store_scatter(v_out, [jnp.full((16,), slot, jnp.int32)],
                       _hsum(ref[...]), mask=lane15)
  pltpu.sync_copy(v_out.at[pl.ds(0, 8)], o_part.at[pl.ds(wid * 8, 8)])


def _tc_bpr_body(pint_ref, ppop_ref, nint_ref, npop_ref, maskf_ref,
                 out_ref):
  pi = pint_ref[...]
  pp = ppop_ref[...]
  ni = nint_ref[...]
  np_ = npop_ref[...]
  mf = maskf_ref[...]
  nmf = 1.0 - mf

  def logsig(x):
    return jnp.log(jax.nn.sigmoid(x))

  loss_int = -jnp.mean(mf * logsig(pi - ni))
  loss_pop = (-jnp.mean(mf * logsig(np_ - pp))
              - jnp.mean(nmf * logsig(pp - np_)))
  loss_total = -jnp.mean(logsig((pi + pp) - (ni + np_)))
  lane = lax.broadcasted_iota(jnp.int32, (1, 8), 1)
  row = jnp.where(lane == 0, loss_int,
                  jnp.where(lane == 1, loss_pop,
                            jnp.where(lane == 2, loss_total, 0.0)))
  out_ref[...] = row


def _tc_combine_body(bpr_ref, part_ref, out_ref):
  bpr = bpr_ref[...]
  ss = jnp.sum(part_ref[...], axis=0, keepdims=True)
  item_mse = ss[0, 0] / (ss[0, 1] * D)
  user_mse = ss[0, 2] / (ss[0, 3] * D)
  loss = (INT_WEIGHT * bpr[0, 0] + POP_WEIGHT * bpr[0, 1] + bpr[0, 2]
          - DIS_PEN * (item_mse + user_mse))
  out_ref[...] = jnp.reshape(loss, (1, 1))


def kernel(user, item_p, item_n, mask, users_int, users_pop, items_int,
           items_pop):
  uf = user.reshape(-1).astype(jnp.int32)
  pf = item_p.reshape(-1).astype(jnp.int32)
  nf = item_n.reshape(-1).astype(jnp.int32)

  (p_int, n_int, p_pop, n_pop, d2p, d2n, d2u, tagi, tagu) = _main_sc(
      uf, pf, nf, users_int, users_pop, items_int, items_pop)

  partials = _reduce_sc(uf, pf, nf, d2p, d2n, d2u, tagi, tagu)

  maskf = mask.reshape(RS).astype(jnp.float32)
  bpr = pl.pallas_call(
      _tc_bpr_body,
      out_shape=jax.ShapeDtypeStruct((1, 8), jnp.float32),
  )(p_int.reshape(RS), p_pop.reshape(RS), n_int.reshape(RS),
    n_pop.reshape(RS), maskf)
  lossm = pl.pallas_call(
      _tc_combine_body,
      out_shape=jax.ShapeDtypeStruct((1, 1), jnp.float32),
  )(bpr, partials.reshape(NW, 8))
  loss = lossm[0, 0]

  return (loss, p_int.reshape(B, L), p_pop.reshape(B, L),
          n_int.reshape(B, L), n_pop.reshape(B, L))
